# baseline (device time: 127414 ns/iter reference)
import jax
import jax.numpy as jnp
from jax import lax
from jax.experimental import pallas as pl
from jax.experimental.pallas import tpu as pltpu

N_DEV = 16
B_LOC = 2
SQ = 128
SKV = 128
HQ_LOC = 4
DH = 64
D_MODEL = 512
HD_LOC = HQ_LOC * DH
BF16 = jnp.bfloat16


def _body(x_ref, wq_ref, k_ref, v_ref, wo_ref, out_ref,
          wq_full, wo_full, q_buf, ctx_buf, acc,
          wq_send, wo_send, wq_recv, wo_recv):
    my = lax.axis_index("i")
    left = lax.rem(my + N_DEV - 1, N_DEV)
    right = lax.rem(my + 1, N_DEV)

    wq_full[my] = wq_ref[...]
    wo_full[my] = wo_ref[...]

    barrier_sem = pltpu.get_barrier_semaphore()
    for nbr in (left, right):
        pl.semaphore_signal(
            barrier_sem, inc=1,
            device_id=(nbr,), device_id_type=pl.DeviceIdType.MESH,
        )
    pl.semaphore_wait(barrier_sem, 2)

    acc[...] = jnp.zeros_like(acc)

    ri = lax.broadcasted_iota(jnp.int32, (SQ, SKV), 0)
    ci = lax.broadcasted_iota(jnp.int32, (SQ, SKV), 1)
    qb = ri // 64
    kb = ci // 64
    mask = jnp.logical_or(qb == kb, (kb % 4) == (qb % 4))

    def compute(o):
        q = jnp.dot(x_ref[...], wq_full[o], preferred_element_type=jnp.float32)
        q_buf[...] = (q * 0.125).astype(BF16)
        for b in range(B_LOC):
            for hh in range(HQ_LOC):
                h_glob = o * HQ_LOC + hh
                qbh = q_buf[b * SQ:(b + 1) * SQ, hh * DH:(hh + 1) * DH]
                kbh = k_ref[b, h_glob]
                s = lax.dot_general(
                    qbh, kbh, (((1,), (1,)), ((), ())),
                    preferred_element_type=jnp.float32,
                )
                s = jnp.where(mask, s, -1e9)
                m = jnp.max(s, axis=1, keepdims=True)
                w = jnp.exp(s - m)
                w = (w / jnp.sum(w, axis=1, keepdims=True)).astype(BF16)
                c = jnp.dot(w, v_ref[b, h_glob],
                            preferred_element_type=jnp.float32)
                ctx_buf[b * SQ:(b + 1) * SQ, hh * DH:(hh + 1) * DH] = (
                    c.astype(BF16))
        acc[...] += jnp.dot(ctx_buf[...], wo_full[o],
                            preferred_element_type=jnp.float32)

    for h in range(N_DEV - 1):
        slot = lax.rem(my + 2 * N_DEV - h, N_DEV)
        rq = pltpu.make_async_remote_copy(
            src_ref=wq_full.at[slot], dst_ref=wq_full.at[slot],
            send_sem=wq_send, recv_sem=wq_recv.at[h],
            device_id=(right,), device_id_type=pl.DeviceIdType.MESH,
        )
        ro = pltpu.make_async_remote_copy(
            src_ref=wo_full.at[slot], dst_ref=wo_full.at[slot],
            send_sem=wo_send, recv_sem=wo_recv.at[h],
            device_id=(right,), device_id_type=pl.DeviceIdType.MESH,
        )
        rq.start()
        ro.start()
        compute(slot)
        rq.wait()
        ro.wait()
    compute(right)

    out_ref[...] = acc[...]


def kernel(x, Wq, K_ext, V_ext, Wo):
    my = lax.axis_index("i")
    x2 = x.reshape(B_LOC * SQ, D_MODEL).astype(BF16)
    wq = Wq.astype(BF16)
    wo = Wo.astype(BF16)
    k_loc = lax.dynamic_slice_in_dim(K_ext, my * B_LOC, B_LOC, axis=0)
    v_loc = lax.dynamic_slice_in_dim(V_ext, my * B_LOC, B_LOC, axis=0)
    k_loc = jnp.transpose(k_loc, (0, 2, 1, 3)).astype(BF16)
    v_loc = jnp.transpose(v_loc, (0, 2, 1, 3)).astype(BF16)

    out2 = pl.pallas_call(
        _body,
        out_shape=jax.ShapeDtypeStruct((B_LOC * SQ, D_MODEL), jnp.float32),
        in_specs=[pl.BlockSpec(memory_space=pltpu.VMEM)] * 5,
        out_specs=pl.BlockSpec(memory_space=pltpu.VMEM),
        scratch_shapes=[
            pltpu.VMEM((N_DEV, D_MODEL, HD_LOC), BF16),
            pltpu.VMEM((N_DEV, HD_LOC, D_MODEL), BF16),
            pltpu.VMEM((B_LOC * SQ, HD_LOC), BF16),
            pltpu.VMEM((B_LOC * SQ, HD_LOC), BF16),
            pltpu.VMEM((B_LOC * SQ, D_MODEL), jnp.float32),
            pltpu.SemaphoreType.DMA,
            pltpu.SemaphoreType.DMA,
            pltpu.SemaphoreType.DMA((N_DEV - 1,)),
            pltpu.SemaphoreType.DMA((N_DEV - 1,)),
        ],
        compiler_params=pltpu.CompilerParams(collective_id=0),
    )(x2, wq, k_loc, v_loc, wo)
    return out2.reshape(B_LOC, SQ, D_MODEL)


# device time: 87671 ns/iter; 1.4533x vs baseline; 1.4533x over previous
import jax
import jax.numpy as jnp
from jax import lax
from jax.experimental import pallas as pl
from jax.experimental.pallas import tpu as pltpu

N_DEV = 16
B_LOC = 2
SQ = 128
SKV = 128
HQ_LOC = 4
DH = 64
D_MODEL = 512
HD_LOC = HQ_LOC * DH
BF16 = jnp.bfloat16


def _body(x_ref, wq_ref, k_ref, v_ref, wo_ref, out_ref,
          wq_full, wo_full, q_buf, ctx_buf, acc,
          wq_send, wo_send, wq_recv, wo_recv):
    my = lax.axis_index("i")
    left = lax.rem(my + N_DEV - 1, N_DEV)
    right = lax.rem(my + 1, N_DEV)

    wq_full[my] = wq_ref[...]
    wo_full[my] = wo_ref[...]

    barrier_sem = pltpu.get_barrier_semaphore()
    for nbr in (left, right):
        pl.semaphore_signal(
            barrier_sem, inc=1,
            device_id=(nbr,), device_id_type=pl.DeviceIdType.MESH,
        )
    pl.semaphore_wait(barrier_sem, 2)

    acc[...] = jnp.zeros_like(acc)

    ri = lax.broadcasted_iota(jnp.int32, (SQ, SKV), 0)
    ci = lax.broadcasted_iota(jnp.int32, (SQ, SKV), 1)
    qb = ri // 64
    kb = ci // 64
    mask = jnp.logical_or(qb == kb, (kb % 4) == (qb % 4))

    def compute(o):
        q = jnp.dot(x_ref[...], wq_full[o], preferred_element_type=jnp.float32)
        q_buf[...] = (q * 0.125).astype(BF16)
        for b in range(B_LOC):
            for hh in range(HQ_LOC):
                h_glob = o * HQ_LOC + hh
                qbh = q_buf[b * SQ:(b + 1) * SQ, hh * DH:(hh + 1) * DH]
                kbh = k_ref[b, h_glob]
                s = lax.dot_general(
                    qbh, kbh, (((1,), (1,)), ((), ())),
                    preferred_element_type=jnp.float32,
                )
                s = jnp.where(mask, s, -1e9)
                m = jnp.max(s, axis=1, keepdims=True)
                w = jnp.exp(s - m)
                w = (w / jnp.sum(w, axis=1, keepdims=True)).astype(BF16)
                c = jnp.dot(w, v_ref[b, h_glob],
                            preferred_element_type=jnp.float32)
                ctx_buf[b * SQ:(b + 1) * SQ, hh * DH:(hh + 1) * DH] = (
                    c.astype(BF16))
        acc[...] += jnp.dot(ctx_buf[...], wo_full[o],
                            preferred_element_type=jnp.float32)

    R_HOPS = N_DEV // 2
    L_HOPS = N_DEV - 1 - R_HOPS
    for h in range(R_HOPS):
        r_slot = lax.rem(my + 2 * N_DEV - h, N_DEV)
        l_slot = lax.rem(my + h, N_DEV)
        rdmas = []
        for (full, send_sem, recv_sem) in (
            (wq_full, wq_send, wq_recv),
            (wo_full, wo_send, wo_recv),
        ):
            rdmas.append(pltpu.make_async_remote_copy(
                src_ref=full.at[r_slot], dst_ref=full.at[r_slot],
                send_sem=send_sem.at[0], recv_sem=recv_sem.at[h],
                device_id=(right,), device_id_type=pl.DeviceIdType.MESH,
            ))
            if h < L_HOPS:
                rdmas.append(pltpu.make_async_remote_copy(
                    src_ref=full.at[l_slot], dst_ref=full.at[l_slot],
                    send_sem=send_sem.at[1],
                    recv_sem=recv_sem.at[R_HOPS + h],
                    device_id=(left,), device_id_type=pl.DeviceIdType.MESH,
                ))
        for r in rdmas:
            r.start()
        if h == 0:
            compute(my)
        else:
            compute(r_slot)
            compute(l_slot)
        for r in rdmas:
            r.wait()
    compute(lax.rem(my + N_DEV // 2, N_DEV))

    out_ref[...] = acc[...]


def kernel(x, Wq, K_ext, V_ext, Wo):
    my = lax.axis_index("i")
    x2 = x.reshape(B_LOC * SQ, D_MODEL).astype(BF16)
    wq = Wq.astype(BF16)
    wo = Wo.astype(BF16)
    k_loc = lax.dynamic_slice_in_dim(K_ext, my * B_LOC, B_LOC, axis=0)
    v_loc = lax.dynamic_slice_in_dim(V_ext, my * B_LOC, B_LOC, axis=0)
    k_loc = jnp.transpose(k_loc, (0, 2, 1, 3)).astype(BF16)
    v_loc = jnp.transpose(v_loc, (0, 2, 1, 3)).astype(BF16)

    out2 = pl.pallas_call(
        _body,
        out_shape=jax.ShapeDtypeStruct((B_LOC * SQ, D_MODEL), jnp.float32),
        in_specs=[pl.BlockSpec(memory_space=pltpu.VMEM)] * 5,
        out_specs=pl.BlockSpec(memory_space=pltpu.VMEM),
        scratch_shapes=[
            pltpu.VMEM((N_DEV, D_MODEL, HD_LOC), BF16),
            pltpu.VMEM((N_DEV, HD_LOC, D_MODEL), BF16),
            pltpu.VMEM((B_LOC * SQ, HD_LOC), BF16),
            pltpu.VMEM((B_LOC * SQ, HD_LOC), BF16),
            pltpu.VMEM((B_LOC * SQ, D_MODEL), jnp.float32),
            pltpu.SemaphoreType.DMA((2,)),
            pltpu.SemaphoreType.DMA((2,)),
            pltpu.SemaphoreType.DMA((N_DEV - 1,)),
            pltpu.SemaphoreType.DMA((N_DEV - 1,)),
        ],
        compiler_params=pltpu.CompilerParams(collective_id=0),
    )(x2, wq, k_loc, v_loc, wo)
    return out2.reshape(B_LOC, SQ, D_MODEL)


# device time: 75480 ns/iter; 1.6880x vs baseline; 1.1615x over previous
import jax
import jax.numpy as jnp
from jax import lax
from jax.experimental import pallas as pl
from jax.experimental.pallas import tpu as pltpu

N_DEV = 16
B_LOC = 2
SQ = 128
SKV = 128
HQ_LOC = 4
DH = 64
D_MODEL = 512
HD_LOC = HQ_LOC * DH
BF16 = jnp.bfloat16


_PI = (0, 1, 5, 9, 13, 14, 10, 6, 2, 3, 7, 11, 15, 12, 8, 4)
_INV_PI = (0, 1, 8, 9, 15, 2, 7, 10, 14, 3, 6, 11, 13, 4, 5, 12)


def _lookup(table, q):
    o = jnp.int32(0)
    for p in range(N_DEV):
        o = jnp.where(q == p, jnp.int32(table[p]), o)
    return o


def _body(x_ref, wq_ref, k_ref, v_ref, wo_ref, out_ref,
          wq_full, wo_full, q_buf, ctx_buf, acc,
          wq_send, wo_send, wq_recv, wo_recv):
    my = lax.axis_index("i")
    p_my = _lookup(_INV_PI, my)
    left = _lookup(_PI, lax.rem(p_my + N_DEV - 1, N_DEV))
    right = _lookup(_PI, lax.rem(p_my + 1, N_DEV))

    wq_full[p_my] = wq_ref[...]
    wo_full[p_my] = wo_ref[...]

    barrier_sem = pltpu.get_barrier_semaphore()
    for nbr in (left, right):
        pl.semaphore_signal(
            barrier_sem, inc=1,
            device_id=(nbr,), device_id_type=pl.DeviceIdType.MESH,
        )
    pl.semaphore_wait(barrier_sem, 2)

    acc[...] = jnp.zeros_like(acc)

    ri = lax.broadcasted_iota(jnp.int32, (SQ, SKV), 0)
    ci = lax.broadcasted_iota(jnp.int32, (SQ, SKV), 1)
    qb = ri // 64
    kb = ci // 64
    mask = jnp.logical_or(qb == kb, (kb % 4) == (qb % 4))

    def compute(slot):
        o = _lookup(_PI, slot)
        q = jnp.dot(x_ref[...], wq_full[slot],
                    preferred_element_type=jnp.float32)
        q_buf[...] = (q * 0.125).astype(BF16)
        for b in range(B_LOC):
            for hh in range(HQ_LOC):
                h_glob = o * HQ_LOC + hh
                qbh = q_buf[b * SQ:(b + 1) * SQ, hh * DH:(hh + 1) * DH]
                kbh = k_ref[b, h_glob]
                s = lax.dot_general(
                    qbh, kbh, (((1,), (1,)), ((), ())),
                    preferred_element_type=jnp.float32,
                )
                s = jnp.where(mask, s, -1e9)
                m = jnp.max(s, axis=1, keepdims=True)
                w = jnp.exp(s - m)
                w = (w / jnp.sum(w, axis=1, keepdims=True)).astype(BF16)
                c = jnp.dot(w, v_ref[b, h_glob],
                            preferred_element_type=jnp.float32)
                ctx_buf[b * SQ:(b + 1) * SQ, hh * DH:(hh + 1) * DH] = (
                    c.astype(BF16))
        acc[...] += jnp.dot(ctx_buf[...], wo_full[slot],
                            preferred_element_type=jnp.float32)

    R_HOPS = N_DEV // 2
    L_HOPS = N_DEV - 1 - R_HOPS
    for h in range(R_HOPS):
        r_slot = lax.rem(p_my + 2 * N_DEV - h, N_DEV)
        l_slot = lax.rem(p_my + h, N_DEV)
        rdmas = []
        for (full, send_sem, recv_sem) in (
            (wq_full, wq_send, wq_recv),
            (wo_full, wo_send, wo_recv),
        ):
            rdmas.append(pltpu.make_async_remote_copy(
                src_ref=full.at[r_slot], dst_ref=full.at[r_slot],
                send_sem=send_sem.at[0], recv_sem=recv_sem.at[h],
                device_id=(right,), device_id_type=pl.DeviceIdType.MESH,
            ))
            if h < L_HOPS:
                rdmas.append(pltpu.make_async_remote_copy(
                    src_ref=full.at[l_slot], dst_ref=full.at[l_slot],
                    send_sem=send_sem.at[1],
                    recv_sem=recv_sem.at[R_HOPS + h],
                    device_id=(left,), device_id_type=pl.DeviceIdType.MESH,
                ))
        for r in rdmas:
            r.start()
        if h == 0:
            compute(p_my)
        else:
            compute(r_slot)
            compute(l_slot)
        for r in rdmas:
            r.wait()
    compute(lax.rem(p_my + N_DEV // 2, N_DEV))

    out_ref[...] = acc[...]


def kernel(x, Wq, K_ext, V_ext, Wo):
    my = lax.axis_index("i")
    x2 = x.reshape(B_LOC * SQ, D_MODEL).astype(BF16)
    wq = Wq.astype(BF16)
    wo = Wo.astype(BF16)
    k_loc = lax.dynamic_slice_in_dim(K_ext, my * B_LOC, B_LOC, axis=0)
    v_loc = lax.dynamic_slice_in_dim(V_ext, my * B_LOC, B_LOC, axis=0)
    k_loc = jnp.transpose(k_loc, (0, 2, 1, 3)).astype(BF16)
    v_loc = jnp.transpose(v_loc, (0, 2, 1, 3)).astype(BF16)

    out2 = pl.pallas_call(
        _body,
        out_shape=jax.ShapeDtypeStruct((B_LOC * SQ, D_MODEL), jnp.float32),
        in_specs=[pl.BlockSpec(memory_space=pltpu.VMEM)] * 5,
        out_specs=pl.BlockSpec(memory_space=pltpu.VMEM),
        scratch_shapes=[
            pltpu.VMEM((N_DEV, D_MODEL, HD_LOC), BF16),
            pltpu.VMEM((N_DEV, HD_LOC, D_MODEL), BF16),
            pltpu.VMEM((B_LOC * SQ, HD_LOC), BF16),
            pltpu.VMEM((B_LOC * SQ, HD_LOC), BF16),
            pltpu.VMEM((B_LOC * SQ, D_MODEL), jnp.float32),
            pltpu.SemaphoreType.DMA((2,)),
            pltpu.SemaphoreType.DMA((2,)),
            pltpu.SemaphoreType.DMA((N_DEV - 1,)),
            pltpu.SemaphoreType.DMA((N_DEV - 1,)),
        ],
        compiler_params=pltpu.CompilerParams(collective_id=0),
    )(x2, wq, k_loc, v_loc, wo)
    return out2.reshape(B_LOC, SQ, D_MODEL)
